# baseline (speedup 1.0000x reference)
import jax
import jax.numpy as jnp
from jax import lax
from jax.experimental import pallas as pl
from jax.experimental.pallas import tpu as pltpu
from jax.experimental.pallas import tpu_sc as plsc

N_NODES = 100000
D = 512
S = 1024
NC = 2
NS = 16
NW = NC * NS
RPT = N_NODES // NW
C = 25
NCHUNK = RPT // C
L = 16
DV = D // L
SEG_PER_TILE = S // NS


def _body(x_hbm, idx_hbm, w_hbm, b_hbm, out_hbm, xb0, xb1, xb2, idxbuf,
          wbuf, bbuf, zbuf, acc, ls0, ls1, ls2, ss0, ss1, ss2):
    c = lax.axis_index("c")
    s = lax.axis_index("s")
    wid = c * NS + s
    row0 = wid * RPT

    pltpu.sync_copy(w_hbm, wbuf)
    pltpu.sync_copy(b_hbm, bbuf)
    pltpu.sync_copy(idx_hbm.at[wid], idxbuf)

    def _zero_row(r, _):
        for j in range(DV):
            zbuf[r, pl.ds(L * j, L)] = jnp.zeros((L,), jnp.float32)
        return 0
    lax.fori_loop(0, SEG_PER_TILE, _zero_row, 0)
    pltpu.sync_copy(zbuf, acc.at[pl.ds(s * SEG_PER_TILE, SEG_PER_TILE)])
    plsc.subcore_barrier()

    bias = bbuf[:]
    ws = [wbuf[pl.ds(L * j, L)] for j in range(DV)]

    def _load(k, xb, sem):
        pltpu.async_copy(x_hbm.at[pl.ds(row0 + k * C, C)], xb, sem)

    def _wait_load(xb, sem):
        pltpu.make_async_copy(x_hbm.at[pl.ds(row0, C)], xb, sem).wait()

    def _scat(k, xb, sem):
        pltpu.async_copy(xb, acc.at[idxbuf.at[k]], sem, add=True)

    def _wait_scat(xb, sem):
        pltpu.make_async_copy(xb, acc.at[idxbuf.at[0]], sem).wait()

    def _row_weight(xb, r):
        accs = [xb[r, pl.ds(L * j, L)] * ws[j] for j in range(4)]
        for j in range(4, DV):
            accs[j % 4] = accs[j % 4] + xb[r, pl.ds(L * j, L)] * ws[j]
        accv = (accs[0] + accs[1]) + (accs[2] + accs[3])
        dot = jnp.sum(accv)
        z = jnp.full((L,), dot, jnp.float32) + bias
        return 1.0 / (1.0 + jnp.exp(-z))

    def _row_scale(xb, r, wv):
        for j in range(DV):
            xb[r, pl.ds(L * j, L)] = xb[r, pl.ds(L * j, L)] * wv

    def _compute(xb):
        def _row_group(r, _):
            rows = [5 * r + u for u in range(5)]
            wvs = [_row_weight(xb, rr) for rr in rows]
            for rr, wv in zip(rows, wvs):
                _row_scale(xb, rr, wv)
            return 0
        lax.fori_loop(0, C // 5, _row_group, 0)

    _load(0, xb0, ls0)
    _load(1, xb1, ls1)

    _wait_load(xb0, ls0)
    _compute(xb0)
    _scat(0, xb0, ss0)
    _load(2, xb2, ls2)

    _wait_load(xb1, ls1)
    _compute(xb1)
    _scat(1, xb1, ss1)
    _wait_scat(xb0, ss0)
    _load(3, xb0, ls0)

    _wait_load(xb2, ls2)
    _compute(xb2)
    _scat(2, xb2, ss2)
    _wait_scat(xb1, ss1)
    _load(4, xb1, ls1)

    def _triple(i, _):
        k = 3 * i
        _wait_load(xb0, ls0)
        _compute(xb0)
        _scat(k, xb0, ss0)
        _wait_scat(xb2, ss2)
        _load(k + 2, xb2, ls2)

        _wait_load(xb1, ls1)
        _compute(xb1)
        _scat(k + 1, xb1, ss1)
        _wait_scat(xb0, ss0)
        _load(k + 3, xb0, ls0)

        _wait_load(xb2, ls2)
        _compute(xb2)
        _scat(k + 2, xb2, ss2)
        _wait_scat(xb1, ss1)
        _load(k + 4, xb1, ls1)
        return 0
    lax.fori_loop(1, (NCHUNK - 2) // 3, _triple, 0)

    _wait_load(xb0, ls0)
    _compute(xb0)
    _scat(NCHUNK - 2, xb0, ss0)
    _wait_scat(xb2, ss2)

    _wait_load(xb1, ls1)
    _compute(xb1)
    _scat(NCHUNK - 1, xb1, ss1)
    _wait_scat(xb0, ss0)
    _wait_scat(xb1, ss1)

    plsc.subcore_barrier()
    pltpu.sync_copy(acc.at[pl.ds(s * SEG_PER_TILE, SEG_PER_TILE)],
                    out_hbm.at[pl.ds(c * S + s * SEG_PER_TILE, SEG_PER_TILE)])


@jax.jit
def _weight_and_sum(x, idx3, w_flat, b16):
    mesh = plsc.VectorSubcoreMesh(core_axis_name="c", subcore_axis_name="s",
                                  num_cores=NC, num_subcores=NS)
    f = pl.kernel(
        _body,
        out_type=jax.ShapeDtypeStruct((NC * S, D), jnp.float32),
        mesh=mesh,
        scratch_types=[
            pltpu.VMEM((C, D), jnp.float32),
            pltpu.VMEM((C, D), jnp.float32),
            pltpu.VMEM((C, D), jnp.float32),
            pltpu.VMEM((NCHUNK, C), jnp.int32),
            pltpu.VMEM((D,), jnp.float32),
            pltpu.VMEM((L,), jnp.float32),
            pltpu.VMEM((SEG_PER_TILE, D), jnp.float32),
            pltpu.VMEM_SHARED((S, D), jnp.float32),
            pltpu.SemaphoreType.DMA,
            pltpu.SemaphoreType.DMA,
            pltpu.SemaphoreType.DMA,
            pltpu.SemaphoreType.DMA,
            pltpu.SemaphoreType.DMA,
            pltpu.SemaphoreType.DMA,
        ],
        compiler_params=pltpu.CompilerParams(use_tc_tiling_on_sc=False,
                                             needs_layout_passes=False),
    )
    partials = f(x, idx3, w_flat, b16)
    return partials[:S] + partials[S:]


def kernel(x, batch, W, b):
    idx3 = batch.reshape(NW, NCHUNK, C)
    w_flat = W.reshape(D)
    b16 = jnp.broadcast_to(b, (L,))
    return _weight_and_sum(x, idx3, w_flat, b16)


# speedup vs baseline: 1.3135x; 1.3135x over previous
import jax
import jax.numpy as jnp
from jax import lax
from jax.experimental import pallas as pl
from jax.experimental.pallas import tpu as pltpu
from jax.experimental.pallas import tpu_sc as plsc

N_NODES = 100000
D = 512
S = 1024
NC = 2
NS = 16
NW = NC * NS
SEGT = S // NW
C = 40
L = 16
DV = D // L
BIGROW = 1 << 30


def _body(x_hbm, batch_hbm, bnd_hbm, w_hbm, b_hbm, out_hbm,
          xb0, xb1, ib0, ib1, bndbuf, wbuf, bbuf, table, ls0, ls1):
    c = lax.axis_index("c")
    s = lax.axis_index("s")
    wid = c * NS + s
    seg_lo = wid * SEGT

    pltpu.sync_copy(w_hbm, wbuf)
    pltpu.sync_copy(b_hbm, bbuf)
    pltpu.sync_copy(bnd_hbm, bndbuf)

    bv = bndbuf[pl.ds(wid, L)]
    rstart = bv[0]
    rend = bv[1]
    astart = (rstart // 8) * 8
    nch = jnp.maximum((rend - astart + C - 1) // C, 0)
    npair = jnp.maximum((nch + 1) // 2, 1)

    def _base(k):
        return pl.multiple_of(jnp.minimum(astart + k * C, N_NODES - C), 8)

    def _lowmask(k):
        return jnp.where(k < nch, jnp.maximum(rstart, astart + k * C), BIGROW)

    def _zero_row(r, _):
        for j in range(DV):
            table[r, pl.ds(L * j, L)] = jnp.zeros((L,), jnp.float32)
        return 0
    lax.fori_loop(0, SEGT, _zero_row, 0)

    bias = bbuf[:]
    ws = [wbuf[pl.ds(L * j, L)] for j in range(DV)]

    def _load(k, xb, ib, sem):
        base = _base(k)
        pltpu.async_copy(x_hbm.at[pl.ds(base, C)], xb, sem)
        pltpu.async_copy(batch_hbm.at[pl.ds(base, C)], ib.at[pl.ds(0, C)], sem)

    def _wait_load(xb, ib, sem):
        pltpu.make_async_copy(x_hbm.at[pl.ds(0, C)], xb, sem).wait()
        pltpu.make_async_copy(batch_hbm.at[pl.ds(0, C)], ib.at[pl.ds(0, C)],
                              sem).wait()

    def _do_row(xb, ib, r, base, lm):
        xs = [xb[r, pl.ds(L * j, L)] for j in range(DV)]
        accs = [xs[j] * ws[j] for j in range(4)]
        for j in range(4, DV):
            accs[j % 4] = accs[j % 4] + xs[j] * ws[j]
        accv = (accs[0] + accs[1]) + (accs[2] + accs[3])
        dot = jnp.sum(accv)
        z = jnp.full((L,), dot, jnp.float32) + bias
        wv = 1.0 / (1.0 + jnp.exp(-z))
        rr = base + r
        ok = jnp.logical_and(rr >= lm, rr < rend)
        wv = wv * jnp.full((L,), jnp.where(ok, 1.0, 0.0), jnp.float32)
        sid = ib[pl.ds(r, L)][0]
        off = jnp.clip(sid - seg_lo, 0, SEGT - 1)
        for j in range(DV):
            plsc.addupdate(table.at[off, pl.ds(L * j, L)], xs[j] * wv)

    def _compute(xb, ib, k):
        base = _base(k)
        lm = _lowmask(k)

        def _row_group(r, _):
            for u in range(5):
                _do_row(xb, ib, 5 * r + u, base, lm)
            return 0
        lax.fori_loop(0, C // 5, _row_group, 0)

    _load(0, xb0, ib0, ls0)
    _load(1, xb1, ib1, ls1)

    def _pair(i, _):
        k = 2 * i
        _wait_load(xb0, ib0, ls0)
        _compute(xb0, ib0, k)
        _load(k + 2, xb0, ib0, ls0)
        _wait_load(xb1, ib1, ls1)
        _compute(xb1, ib1, k + 1)
        _load(k + 3, xb1, ib1, ls1)
        return 0
    lax.fori_loop(0, npair, _pair, 0)

    _wait_load(xb0, ib0, ls0)
    _wait_load(xb1, ib1, ls1)

    out0 = pl.multiple_of(wid * SEGT, 8)
    pltpu.sync_copy(table, out_hbm.at[pl.ds(out0, SEGT)])


@jax.jit
def _weight_and_sum(x, batch, bnd, w_flat, b16):
    mesh = plsc.VectorSubcoreMesh(core_axis_name="c", subcore_axis_name="s",
                                  num_cores=NC, num_subcores=NS)
    f = pl.kernel(
        _body,
        out_type=jax.ShapeDtypeStruct((S, D), jnp.float32),
        mesh=mesh,
        scratch_types=[
            pltpu.VMEM((C, D), jnp.float32),
            pltpu.VMEM((C, D), jnp.float32),
            pltpu.VMEM((C + L,), jnp.int32),
            pltpu.VMEM((C + L,), jnp.int32),
            pltpu.VMEM((48,), jnp.int32),
            pltpu.VMEM((D,), jnp.float32),
            pltpu.VMEM((L,), jnp.float32),
            pltpu.VMEM((SEGT, D), jnp.float32),
            pltpu.SemaphoreType.DMA,
            pltpu.SemaphoreType.DMA,
        ],
        compiler_params=pltpu.CompilerParams(needs_layout_passes=False),
    )
    return f(x, batch, bnd, w_flat, b16)


def kernel(x, batch, W, b):
    edges = jnp.arange(0, S + 1, SEGT, dtype=jnp.int32)
    bnd = jnp.searchsorted(batch, edges, side="left").astype(jnp.int32)
    bnd48 = jnp.concatenate([bnd, jnp.zeros((15,), jnp.int32)])
    w_flat = W.reshape(D)
    b16 = jnp.broadcast_to(b, (L,))
    return _weight_and_sum(x, batch, bnd48, w_flat, b16)
